# baseline (speedup 1.0000x reference)
import jax
import jax.numpy as jnp
from jax import lax
from jax.experimental import pallas as pl
from jax.experimental.pallas import tpu as pltpu
from jax.experimental.pallas import tpu_sc as plsc

N = 10000
K = 32
C0 = 128
C1 = 64
NKP = 15
EXT = 0.12
B = 4
SEG = N // B

N2 = 10240
E2 = N2 * K

QB = 320
GRID = N2 // QB
G = 8
NG = QB // G
GK = G * K

NC = 2
NS = 16
NW = NC * NS
EPW = E2 // NW

_HI = jax.lax.Precision.HIGHEST


def _lrelu(x):
    return jnp.where(x >= 0, x, 0.1 * x)


def _sc_pipe(table, outs, idx_v, bufs, gsems, ssems, rb, base, nch,
             extract=None):
    depth = len(bufs)
    lead = depth - 1
    gh = [None] * depth
    sh = [None] * depth
    for i in range(nch + lead):
        if i < nch:
            b = i % depth
            if sh[b] is not None:
                for h in sh[b]:
                    h.wait()
            gh[b] = pltpu.async_copy(
                table.at[idx_v.at[pl.ds(i * rb, rb)]], bufs[b], gsems[b])
        j = i - lead
        if 0 <= j:
            bj = j % depth
            gh[bj].wait()
            if extract is not None:
                extract(bufs[bj], j)
            sh[bj] = []
            for out, lo, ln in outs:
                sh[bj].append(pltpu.async_copy(
                    bufs[bj].at[:, pl.ds(lo, ln)],
                    out.at[pl.ds(base + j * rb, rb)], ssems[bj]))
    for hs in sh:
        if hs is not None:
            for h in hs:
                h.wait()


_RB = 256
_RB0 = 128


def _sc_gather0_body(ftab_hbm, idx_hbm, g0_hbm,
                     idx_v, b0, b1, b2, gs0, gs1, gs2, ss0, ss1, ss2):
    wid = lax.axis_index("s") * NC + lax.axis_index("c")
    base = wid * EPW
    pltpu.sync_copy(idx_hbm.at[pl.ds(base, EPW)], idx_v)
    _sc_pipe(ftab_hbm, [(g0_hbm, 0, 2 * C0)], idx_v, [b0, b1, b2],
             [gs0, gs1, gs2], [ss0, ss1, ss2], _RB0, base, EPW // _RB0)


def _sc_gather1_body(x_hbm, idx_hbm, g1_hbm,
                     idx_v, b0, b1, b2, gs0, gs1, gs2, ss0, ss1, ss2):
    wid = lax.axis_index("s") * NC + lax.axis_index("c")
    base = wid * EPW
    pltpu.sync_copy(idx_hbm.at[pl.ds(base, EPW)], idx_v)
    _sc_pipe(x_hbm, [(g1_hbm, 0, C0)], idx_v, [b0, b1, b2], [gs0, gs1, gs2],
             [ss0, ss1, ss2], _RB, base, EPW // _RB)


def _gather0(ftab, ef):
    mesh = plsc.VectorSubcoreMesh(core_axis_name="c", subcore_axis_name="s")
    return pl.kernel(
        _sc_gather0_body,
        mesh=mesh,
        out_type=jax.ShapeDtypeStruct((E2, 2 * C0), jnp.float32),
        scratch_types=[
            pltpu.VMEM((EPW,), jnp.int32),
            pltpu.VMEM((_RB0, 2 * C0), jnp.float32),
            pltpu.VMEM((_RB0, 2 * C0), jnp.float32),
            pltpu.VMEM((_RB0, 2 * C0), jnp.float32),
            pltpu.SemaphoreType.DMA,
            pltpu.SemaphoreType.DMA,
            pltpu.SemaphoreType.DMA,
            pltpu.SemaphoreType.DMA,
            pltpu.SemaphoreType.DMA,
            pltpu.SemaphoreType.DMA,
        ],
    )(ftab, ef)


def _gather1(x, ef):
    mesh = plsc.VectorSubcoreMesh(core_axis_name="c", subcore_axis_name="s")
    return pl.kernel(
        _sc_gather1_body,
        mesh=mesh,
        out_type=jax.ShapeDtypeStruct((E2, C0), jnp.float32),
        scratch_types=[
            pltpu.VMEM((EPW,), jnp.int32),
            pltpu.VMEM((_RB, C0), jnp.float32),
            pltpu.VMEM((_RB, C0), jnp.float32),
            pltpu.VMEM((_RB, C0), jnp.float32),
            pltpu.SemaphoreType.DMA,
            pltpu.SemaphoreType.DMA,
            pltpu.SemaphoreType.DMA,
            pltpu.SemaphoreType.DMA,
            pltpu.SemaphoreType.DMA,
            pltpu.SemaphoreType.DMA,
        ],
    )(x, ef)


def _kpconv_weights_tiled(nxt, nyt, nzt, qxt, qyt, qzt, kp_ref):
    dx = nxt - qxt
    dy = nyt - qyt
    dz = nzt - qzt
    d2 = dx * dx + dy * dy + dz * dz
    ws = []
    for p in range(NKP):
        kx = kp_ref[p, 0]
        ky = kp_ref[p, 1]
        kz = kp_ref[p, 2]
        t = d2 - 2.0 * (dx * kx + dy * ky + dz * kz) + (kx * kx + ky * ky + kz * kz)
        w = jnp.maximum(0.0, 1.0 - jnp.sqrt(t + 1e-12) * (1.0 / EXT))
        ws.append(w)
    return ws


def _group_mask():
    rows = jax.lax.broadcasted_iota(jnp.int32, (G, GK), 0)
    cols = jax.lax.broadcasted_iota(jnp.int32, (G, GK), 1)
    return jnp.where(rows == cols // K, 1.0, 0.0).astype(jnp.float32)


def _kpconv_mxu(ws, g_ref, wrows_ref):
    mask = _group_mask()
    for gi in range(NG):
        bands = []
        for p in range(NKP):
            row = ws[p][gi:gi + 1, :]
            bands.append(jnp.broadcast_to(row, (G, GK)) * mask)
        a = jnp.concatenate(bands, axis=0)
        gg = g_ref[gi * GK:(gi + 1) * GK, :]
        o = jnp.dot(a, gg, preferred_element_type=jnp.float32)
        for p in range(NKP):
            wrows_ref[p, gi * G:(gi + 1) * G, :] = o[p * G:(p + 1) * G, :]


def _apply_kernel_weights(wrows_ref, w_ref, d):
    acc = jnp.zeros((QB, d), jnp.float32)
    for p in range(NKP):
        acc = acc + jnp.dot(wrows_ref[p], w_ref[p],
                            preferred_element_type=jnp.float32)
    return acc


def _block0_body(kp_ref, nxt_ref, nyt_ref, nzt_ref, qxt_ref, qyt_ref, qzt_ref,
                 g_ref, w0_ref, wu1_ref, f0_ref, x_ref, wrows_ref):
    ws = _kpconv_weights_tiled(nxt_ref[...], nyt_ref[...], nzt_ref[...],
                               qxt_ref[...], qyt_ref[...], qzt_ref[...],
                               kp_ref)
    _kpconv_mxu(ws, g_ref, wrows_ref)
    out = _apply_kernel_weights(wrows_ref, w0_ref, C0)
    f0 = _lrelu(out)
    f0_ref[...] = f0
    x_ref[...] = _lrelu(jnp.dot(f0, wu1_ref[...],
                                preferred_element_type=jnp.float32))


def _block1_body(kp_ref, nxt_ref, nyt_ref, nzt_ref, qxt_ref, qyt_ref, qzt_ref,
                 g_ref, wk1_ref, wu2_ref, f0_ref, out_ref, wrows_ref):
    ws = _kpconv_weights_tiled(nxt_ref[...], nyt_ref[...], nzt_ref[...],
                               qxt_ref[...], qyt_ref[...], qzt_ref[...],
                               kp_ref)
    _kpconv_mxu(ws, g_ref, wrows_ref)
    x = _lrelu(_apply_kernel_weights(wrows_ref, wk1_ref, C1))
    x = jnp.dot(x, wu2_ref[...], preferred_element_type=jnp.float32)
    out_ref[...] = _lrelu(x + f0_ref[...])


def _pool_body(f_ref, out_ref):
    rows = jax.lax.broadcasted_iota(jnp.int32, (8, N2), 0)
    cols = jax.lax.broadcasted_iota(jnp.int32, (8, N2), 1)
    sel = jnp.where(rows == cols // SEG, 1.0 / SEG, 0.0).astype(jnp.float32)
    out_ref[...] = jnp.dot(sel, f_ref[...], preferred_element_type=jnp.float32,
                           precision=_HI)


def _tile_spec():
    return pl.BlockSpec((NG, GK), lambda i: (i, 0))


def _full_spec(shape):
    return pl.BlockSpec(shape, lambda i: tuple(0 for _ in shape))


def _run_block0(K_points0, nxt, nyt, nzt, qxt, qyt, qzt, g0, W0, Wu1):
    return pl.pallas_call(
        _block0_body,
        grid=(GRID,),
        in_specs=[
            pl.BlockSpec(memory_space=pltpu.SMEM),
            _tile_spec(), _tile_spec(), _tile_spec(),
            _tile_spec(), _tile_spec(), _tile_spec(),
            pl.BlockSpec((QB * K, C0), lambda i: (i, 0)),
            _full_spec((NKP, C0, C0)),
            _full_spec((C0, C0)),
        ],
        out_specs=[
            pl.BlockSpec((QB, C0), lambda i: (i, 0)),
            pl.BlockSpec((QB, C0), lambda i: (i, 0)),
        ],
        out_shape=[
            jax.ShapeDtypeStruct((N2, C0), jnp.float32),
            jax.ShapeDtypeStruct((N2, C0), jnp.float32),
        ],
        scratch_shapes=[pltpu.VMEM((NKP, QB, C0), jnp.float32)],
    )(K_points0, nxt, nyt, nzt, qxt, qyt, qzt, g0, W0, Wu1)


def _run_block1(K_points1, nxt, nyt, nzt, qxt, qyt, qzt, g1, Wk1, Wu2, f0):
    return pl.pallas_call(
        _block1_body,
        grid=(GRID,),
        in_specs=[
            pl.BlockSpec(memory_space=pltpu.SMEM),
            _tile_spec(), _tile_spec(), _tile_spec(),
            _tile_spec(), _tile_spec(), _tile_spec(),
            pl.BlockSpec((QB * K, C0), lambda i: (i, 0)),
            _full_spec((NKP, C0, C1)),
            _full_spec((C1, C0)),
            pl.BlockSpec((QB, C0), lambda i: (i, 0)),
        ],
        out_specs=pl.BlockSpec((QB, C0), lambda i: (i, 0)),
        out_shape=jax.ShapeDtypeStruct((N2, C0), jnp.float32),
        scratch_shapes=[pltpu.VMEM((NKP, QB, C0), jnp.float32)],
    )(K_points1, nxt, nyt, nzt, qxt, qyt, qzt, g1, Wk1, Wu2, f0)


def _run_pool(f):
    return pl.pallas_call(
        _pool_body,
        grid=(1,),
        in_specs=[_full_spec((N2, C0))],
        out_specs=pl.BlockSpec((8, C0), lambda i: (0, 0)),
        out_shape=jax.ShapeDtypeStruct((8, C0), jnp.float32),
    )(f)


def kernel(points, neighbors, features, stack_lengths, K_points0, W0, Wu1,
           K_points1, Wk1, Wu2):
    del stack_lengths
    pad = N2 - N
    ef = jnp.pad(neighbors, ((0, pad), (0, 0))).reshape(-1).astype(jnp.int32)
    ftab = jnp.pad(jnp.concatenate([features, points], axis=1),
                   ((0, 0), (0, 2 * C0 - C0 - 3)))

    g0 = _gather0(ftab, ef)

    nxt = g0[:, C0 + 0].reshape(N2 // G, GK)
    nyt = g0[:, C0 + 1].reshape(N2 // G, GK)
    nzt = g0[:, C0 + 2].reshape(N2 // G, GK)
    p2 = jnp.pad(points, ((0, pad), (0, 0)))
    qt = jnp.broadcast_to(p2.reshape(N2 // G, G, 1, 3), (N2 // G, G, K, 3))
    qxt = qt[..., 0].reshape(N2 // G, GK)
    qyt = qt[..., 1].reshape(N2 // G, GK)
    qzt = qt[..., 2].reshape(N2 // G, GK)

    wu1p = jnp.pad(Wu1, ((0, 0), (0, C0 - C1)))
    wk1p = jnp.pad(Wk1, ((0, 0), (0, C0 - C1), (0, 0)))
    f0, x = _run_block0(K_points0, nxt, nyt, nzt, qxt, qyt, qzt, g0, W0, wu1p)
    g1 = _gather1(x, ef)
    f = _run_block1(K_points1, nxt, nyt, nzt, qxt, qyt, qzt, g1, wk1p, Wu2, f0)
    pooled8 = _run_pool(f)
    return (f[:N], pooled8[:B])


# speedup vs baseline: 1.0011x; 1.0011x over previous
import jax
import jax.numpy as jnp
from jax import lax
from jax.experimental import pallas as pl
from jax.experimental.pallas import tpu as pltpu
from jax.experimental.pallas import tpu_sc as plsc

N = 10000
K = 32
C0 = 128
C1 = 64
NKP = 15
EXT = 0.12
B = 4
SEG = N // B

N2 = 10240
E2 = N2 * K

QB = 320
GRID = N2 // QB
G = 8
NG = QB // G
GK = G * K

NC = 2
NS = 16
NW = NC * NS
EPW = E2 // NW

_HI = jax.lax.Precision.HIGHEST


def _lrelu(x):
    return jnp.where(x >= 0, x, 0.1 * x)


def _sc_pipe(table, outs, idx_v, bufs, gsems, ssems, rb, base, nch,
             extract=None):
    depth = len(bufs)
    lead = depth - 1
    gh = [None] * depth
    sh = [None] * depth
    for i in range(nch + lead):
        if i < nch:
            b = i % depth
            if sh[b] is not None:
                for h in sh[b]:
                    h.wait()
            gh[b] = pltpu.async_copy(
                table.at[idx_v.at[pl.ds(i * rb, rb)]], bufs[b], gsems[b])
        j = i - lead
        if 0 <= j:
            bj = j % depth
            gh[bj].wait()
            if extract is not None:
                extract(bufs[bj], j)
            sh[bj] = []
            for out, lo, ln in outs:
                sh[bj].append(pltpu.async_copy(
                    bufs[bj].at[:, pl.ds(lo, ln)],
                    out.at[pl.ds(base + j * rb, rb)], ssems[bj]))
    for hs in sh:
        if hs is not None:
            for h in hs:
                h.wait()


_RB = 256
_RB0 = 128


def _sc_gather0_body(ftab_hbm, idx_hbm, g0_hbm,
                     idx_v, b0, b1, b2, gs0, gs1, gs2, ss0, ss1, ss2):
    wid = lax.axis_index("s") * NC + lax.axis_index("c")
    base = wid * EPW
    pltpu.sync_copy(idx_hbm.at[pl.ds(base, EPW)], idx_v)
    _sc_pipe(ftab_hbm, [(g0_hbm, 0, 2 * C0)], idx_v, [b0, b1, b2],
             [gs0, gs1, gs2], [ss0, ss1, ss2], _RB0, base, EPW // _RB0)


def _sc_gather1_body(x_hbm, idx_hbm, g1_hbm,
                     idx_v, b0, b1, b2, gs0, gs1, gs2, ss0, ss1, ss2):
    wid = lax.axis_index("s") * NC + lax.axis_index("c")
    base = wid * EPW
    pltpu.sync_copy(idx_hbm.at[pl.ds(base, EPW)], idx_v)
    _sc_pipe(x_hbm, [(g1_hbm, 0, C0)], idx_v, [b0, b1, b2], [gs0, gs1, gs2],
             [ss0, ss1, ss2], _RB0, base, EPW // _RB0)


def _gather0(ftab, ef):
    mesh = plsc.VectorSubcoreMesh(core_axis_name="c", subcore_axis_name="s")
    return pl.kernel(
        _sc_gather0_body,
        mesh=mesh,
        out_type=jax.ShapeDtypeStruct((E2, 2 * C0), jnp.float32),
        scratch_types=[
            pltpu.VMEM((EPW,), jnp.int32),
            pltpu.VMEM((_RB0, 2 * C0), jnp.float32),
            pltpu.VMEM((_RB0, 2 * C0), jnp.float32),
            pltpu.VMEM((_RB0, 2 * C0), jnp.float32),
            pltpu.SemaphoreType.DMA,
            pltpu.SemaphoreType.DMA,
            pltpu.SemaphoreType.DMA,
            pltpu.SemaphoreType.DMA,
            pltpu.SemaphoreType.DMA,
            pltpu.SemaphoreType.DMA,
        ],
    )(ftab, ef)


def _gather1(x, ef):
    mesh = plsc.VectorSubcoreMesh(core_axis_name="c", subcore_axis_name="s")
    return pl.kernel(
        _sc_gather1_body,
        mesh=mesh,
        out_type=jax.ShapeDtypeStruct((E2, C0), jnp.float32),
        scratch_types=[
            pltpu.VMEM((EPW,), jnp.int32),
            pltpu.VMEM((_RB0, C0), jnp.float32),
            pltpu.VMEM((_RB0, C0), jnp.float32),
            pltpu.VMEM((_RB0, C0), jnp.float32),
            pltpu.SemaphoreType.DMA,
            pltpu.SemaphoreType.DMA,
            pltpu.SemaphoreType.DMA,
            pltpu.SemaphoreType.DMA,
            pltpu.SemaphoreType.DMA,
            pltpu.SemaphoreType.DMA,
        ],
    )(x, ef)


def _kpconv_weights_tiled(nxt, nyt, nzt, qxt, qyt, qzt, kp_ref):
    dx = nxt - qxt
    dy = nyt - qyt
    dz = nzt - qzt
    d2 = dx * dx + dy * dy + dz * dz
    ws = []
    for p in range(NKP):
        kx = kp_ref[p, 0]
        ky = kp_ref[p, 1]
        kz = kp_ref[p, 2]
        t = d2 - 2.0 * (dx * kx + dy * ky + dz * kz) + (kx * kx + ky * ky + kz * kz)
        w = jnp.maximum(0.0, 1.0 - jnp.sqrt(t + 1e-12) * (1.0 / EXT))
        ws.append(w)
    return ws


def _group_mask():
    rows = jax.lax.broadcasted_iota(jnp.int32, (G, GK), 0)
    cols = jax.lax.broadcasted_iota(jnp.int32, (G, GK), 1)
    return jnp.where(rows == cols // K, 1.0, 0.0).astype(jnp.float32)


def _kpconv_mxu(ws, g_ref, wrows_ref):
    mask = _group_mask()
    for gi in range(NG):
        bands = []
        for p in range(NKP):
            row = ws[p][gi:gi + 1, :]
            bands.append(jnp.broadcast_to(row, (G, GK)) * mask)
        a = jnp.concatenate(bands, axis=0)
        gg = g_ref[gi * GK:(gi + 1) * GK, :]
        o = jnp.dot(a, gg, preferred_element_type=jnp.float32)
        for p in range(NKP):
            wrows_ref[p, gi * G:(gi + 1) * G, :] = o[p * G:(p + 1) * G, :]


def _apply_kernel_weights(wrows_ref, w_ref, d):
    acc = jnp.zeros((QB, d), jnp.float32)
    for p in range(NKP):
        acc = acc + jnp.dot(wrows_ref[p], w_ref[p],
                            preferred_element_type=jnp.float32)
    return acc


def _block0_body(kp_ref, nxt_ref, nyt_ref, nzt_ref, qxt_ref, qyt_ref, qzt_ref,
                 g_ref, w0_ref, wu1_ref, f0_ref, x_ref, wrows_ref):
    ws = _kpconv_weights_tiled(nxt_ref[...], nyt_ref[...], nzt_ref[...],
                               qxt_ref[...], qyt_ref[...], qzt_ref[...],
                               kp_ref)
    _kpconv_mxu(ws, g_ref, wrows_ref)
    out = _apply_kernel_weights(wrows_ref, w0_ref, C0)
    f0 = _lrelu(out)
    f0_ref[...] = f0
    x_ref[...] = _lrelu(jnp.dot(f0, wu1_ref[...],
                                preferred_element_type=jnp.float32))


def _block1_body(kp_ref, nxt_ref, nyt_ref, nzt_ref, qxt_ref, qyt_ref, qzt_ref,
                 g_ref, wk1_ref, wu2_ref, f0_ref, out_ref, wrows_ref):
    ws = _kpconv_weights_tiled(nxt_ref[...], nyt_ref[...], nzt_ref[...],
                               qxt_ref[...], qyt_ref[...], qzt_ref[...],
                               kp_ref)
    _kpconv_mxu(ws, g_ref, wrows_ref)
    x = _lrelu(_apply_kernel_weights(wrows_ref, wk1_ref, C1))
    x = jnp.dot(x, wu2_ref[...], preferred_element_type=jnp.float32)
    out_ref[...] = _lrelu(x + f0_ref[...])


def _pool_body(f_ref, out_ref):
    rows = jax.lax.broadcasted_iota(jnp.int32, (8, N2), 0)
    cols = jax.lax.broadcasted_iota(jnp.int32, (8, N2), 1)
    sel = jnp.where(rows == cols // SEG, 1.0 / SEG, 0.0).astype(jnp.float32)
    out_ref[...] = jnp.dot(sel, f_ref[...], preferred_element_type=jnp.float32,
                           precision=_HI)


def _tile_spec():
    return pl.BlockSpec((NG, GK), lambda i: (i, 0))


def _full_spec(shape):
    return pl.BlockSpec(shape, lambda i: tuple(0 for _ in shape))


def _run_block0(K_points0, nxt, nyt, nzt, qxt, qyt, qzt, g0, W0, Wu1):
    return pl.pallas_call(
        _block0_body,
        grid=(GRID,),
        in_specs=[
            pl.BlockSpec(memory_space=pltpu.SMEM),
            _tile_spec(), _tile_spec(), _tile_spec(),
            _tile_spec(), _tile_spec(), _tile_spec(),
            pl.BlockSpec((QB * K, C0), lambda i: (i, 0)),
            _full_spec((NKP, C0, C0)),
            _full_spec((C0, C0)),
        ],
        out_specs=[
            pl.BlockSpec((QB, C0), lambda i: (i, 0)),
            pl.BlockSpec((QB, C0), lambda i: (i, 0)),
        ],
        out_shape=[
            jax.ShapeDtypeStruct((N2, C0), jnp.float32),
            jax.ShapeDtypeStruct((N2, C0), jnp.float32),
        ],
        scratch_shapes=[pltpu.VMEM((NKP, QB, C0), jnp.float32)],
    )(K_points0, nxt, nyt, nzt, qxt, qyt, qzt, g0, W0, Wu1)


def _run_block1(K_points1, nxt, nyt, nzt, qxt, qyt, qzt, g1, Wk1, Wu2, f0):
    return pl.pallas_call(
        _block1_body,
        grid=(GRID,),
        in_specs=[
            pl.BlockSpec(memory_space=pltpu.SMEM),
            _tile_spec(), _tile_spec(), _tile_spec(),
            _tile_spec(), _tile_spec(), _tile_spec(),
            pl.BlockSpec((QB * K, C0), lambda i: (i, 0)),
            _full_spec((NKP, C0, C1)),
            _full_spec((C1, C0)),
            pl.BlockSpec((QB, C0), lambda i: (i, 0)),
        ],
        out_specs=pl.BlockSpec((QB, C0), lambda i: (i, 0)),
        out_shape=jax.ShapeDtypeStruct((N2, C0), jnp.float32),
        scratch_shapes=[pltpu.VMEM((NKP, QB, C0), jnp.float32)],
    )(K_points1, nxt, nyt, nzt, qxt, qyt, qzt, g1, Wk1, Wu2, f0)


def _run_pool(f):
    return pl.pallas_call(
        _pool_body,
        grid=(1,),
        in_specs=[_full_spec((N2, C0))],
        out_specs=pl.BlockSpec((8, C0), lambda i: (0, 0)),
        out_shape=jax.ShapeDtypeStruct((8, C0), jnp.float32),
    )(f)


def kernel(points, neighbors, features, stack_lengths, K_points0, W0, Wu1,
           K_points1, Wk1, Wu2):
    del stack_lengths
    pad = N2 - N
    ef = jnp.pad(neighbors, ((0, pad), (0, 0))).reshape(-1).astype(jnp.int32)
    ftab = jnp.pad(jnp.concatenate([features, points], axis=1),
                   ((0, 0), (0, 2 * C0 - C0 - 3)))

    g0 = _gather0(ftab, ef)

    nxt = g0[:, C0 + 0].reshape(N2 // G, GK)
    nyt = g0[:, C0 + 1].reshape(N2 // G, GK)
    nzt = g0[:, C0 + 2].reshape(N2 // G, GK)
    p2 = jnp.pad(points, ((0, pad), (0, 0)))
    qt = jnp.broadcast_to(p2.reshape(N2 // G, G, 1, 3), (N2 // G, G, K, 3))
    qxt = qt[..., 0].reshape(N2 // G, GK)
    qyt = qt[..., 1].reshape(N2 // G, GK)
    qzt = qt[..., 2].reshape(N2 // G, GK)

    wu1p = jnp.pad(Wu1, ((0, 0), (0, C0 - C1)))
    wk1p = jnp.pad(Wk1, ((0, 0), (0, C0 - C1), (0, 0)))
    f0, x = _run_block0(K_points0, nxt, nyt, nzt, qxt, qyt, qzt, g0, W0, wu1p)
    g1 = _gather1(x, ef)
    f = _run_block1(K_points1, nxt, nyt, nzt, qxt, qyt, qzt, g1, wk1p, Wu2, f0)
    pooled8 = _run_pool(f)
    return (f[:N], pooled8[:B])
